# baseline (speedup 1.0000x reference)
import functools

import jax
import jax.numpy as jnp
from jax import lax
from jax.experimental import pallas as pl
from jax.experimental.pallas import tpu as pltpu
from jax.experimental.pallas import tpu_sc as plsc

ITEM_NUM = 1000000
HIDDEN = 64
MAXLEN = 200
BATCH = 4096

SCALE = float(HIDDEN) ** 0.5

NUM_CORES = 2
NUM_SUBCORES = 16
NW = NUM_CORES * NUM_SUBCORES
SEQ_PER_W = BATCH // NW
PER_W = SEQ_PER_W * MAXLEN
NB = 4
LA = 2

SPLIT_A = 96
SPLIT_B = MAXLEN - SPLIT_A
BR_ROWS = BATCH * SPLIT_B

_mesh = plsc.VectorSubcoreMesh(core_axis_name="c", subcore_axis_name="s")


@functools.partial(
    pl.kernel,
    mesh=_mesh,
    compiler_params=pltpu.CompilerParams(use_tc_tiling_on_sc=False),
    out_type=jax.ShapeDtypeStruct((BR_ROWS, 2 * HIDDEN), jnp.float32),
    scratch_types=(
        [pltpu.VMEM((SEQ_PER_W, MAXLEN), jnp.float32),
         pltpu.VMEM((SEQ_PER_W, MAXLEN), jnp.int32),
         pltpu.VMEM((MAXLEN, HIDDEN), jnp.float32)]
        + [pltpu.VMEM((MAXLEN, HIDDEN), jnp.float32) for _ in range(NB)]
        + [pltpu.SemaphoreType.DMA for _ in range(2 * NB)]
    ),
)
def _embed_kernel(idx_hbm, table_hbm, pos_hbm, out_hbm, idxf_v, idx_v, pos_v,
                  b0, b1, b2, b3, g0, g1, g2, g3, s0, s1, s2, s3):
    bufs = [b0, b1, b2, b3]
    gsem = [g0, g1, g2, g3]
    ssem = [s0, s1, s2, s3]
    wid = lax.axis_index("s") * NUM_CORES + lax.axis_index("c")
    seq_base = wid * SEQ_PER_W
    pltpu.sync_copy(idx_hbm.at[pl.ds(seq_base, SEQ_PER_W)], idxf_v)
    pltpu.sync_copy(pos_hbm, pos_v)

    _starts = [k * 16 for k in range(MAXLEN // 16)] + [MAXLEN - 16]

    def conv_seq(s, carry):
        for st in _starts:
            sl = pl.ds(st, 16)
            idx_v[s, sl] = idxf_v[s, sl].astype(jnp.int32)
        return carry

    lax.fori_loop(0, SEQ_PER_W, conv_seq, 0)

    def g_start(c, b):
        pltpu.async_copy(table_hbm.at[idx_v.at[c]], bufs[b], gsem[b])

    def g_wait(c, b):
        pltpu.make_async_copy(table_hbm.at[idx_v.at[c]], bufs[b],
                              gsem[b]).wait()

    def s_parts(c, b):
        row0 = (seq_base + c) * SPLIT_B
        yield (bufs[b].at[pl.ds(0, SPLIT_A)],
               out_hbm.at[pl.ds(row0, SPLIT_A), pl.ds(0, HIDDEN)])
        yield (bufs[b].at[pl.ds(SPLIT_A, SPLIT_B)],
               out_hbm.at[pl.ds(row0, SPLIT_B), pl.ds(HIDDEN, HIDDEN)])

    def s_start(c, b):
        for src, dst in s_parts(c, b):
            pltpu.async_copy(src, dst, ssem[b])

    def s_wait(c, b):
        for src, dst in s_parts(c, b):
            pltpu.make_async_copy(src, dst, ssem[b]).wait()

    for c in range(LA):
        g_start(c, c % NB)

    def outer(o, carry):
        for b in range(NB):
            c = o * NB + b
            g_wait(c, b)

            def row(j, carry2):
                for h in range(HIDDEN // 16):
                    sl = pl.ds(h * 16, 16)
                    bufs[b][j, sl] = bufs[b][j, sl] * SCALE + pos_v[j, sl]
                return carry2

            lax.fori_loop(0, MAXLEN, row, 0, unroll=4)
            s_start(c, b)
            b3 = (b + LA + 1) % NB

            @pl.when(c >= 1)
            def _():
                s_wait(c - 1, b3)
            cg = c + LA
            bg = (b + LA) % NB

            @pl.when(cg < SEQ_PER_W)
            def _():
                g_start(cg, bg)
        return carry

    lax.fori_loop(0, SEQ_PER_W // NB, outer, 0)
    c_last = SEQ_PER_W - 1
    s_wait(c_last, c_last % NB)


CP = 2


@functools.partial(
    pl.kernel,
    mesh=_mesh,
    compiler_params=pltpu.CompilerParams(needs_layout_passes=False),
    out_type=jax.ShapeDtypeStruct((BATCH, HIDDEN, MAXLEN), jnp.float32),
    scratch_types=(
        [pltpu.VMEM((SPLIT_B, 2 * HIDDEN), jnp.float32) for _ in range(CP)]
        + [pltpu.VMEM((HIDDEN, MAXLEN), jnp.float32) for _ in range(CP)]
        + [pltpu.SemaphoreType.DMA for _ in range(2 * CP)]
    ),
)
def _retile_out(br_hbm, out_hbm, vi0, vi1, vo0, vo1, i0, i1, o0, o1):
    vin = [vi0, vi1]
    vout = [vo0, vo1]
    isem = [i0, i1]
    osem = [o0, o1]
    wid = lax.axis_index("s") * NUM_CORES + lax.axis_index("c")
    seq_base = wid * SEQ_PER_W

    def in_pair(c, p):
        return (br_hbm.at[pl.ds((seq_base + c) * SPLIT_B, SPLIT_B)], vin[p])

    def out_pair(c, p):
        return (vout[p], out_hbm.at[seq_base + c])

    src, dst = in_pair(0, 0)
    pltpu.async_copy(src, dst, isem[0])

    def outer(o, carry):
        for p in range(CP):
            c = o * CP + p
            src, dst = in_pair(c, p)
            pltpu.make_async_copy(src, dst, isem[p]).wait()
            cn = c + 1

            @pl.when(cn < SEQ_PER_W)
            def _():
                s2, d2 = in_pair(cn, (p + 1) % CP)
                pltpu.async_copy(s2, d2, isem[(p + 1) % CP])

            @pl.when(c >= CP)
            def _():
                s3, d3 = out_pair(c - CP, p)
                pltpu.make_async_copy(s3, d3, osem[p]).wait()

            iota16 = jnp.arange(16, dtype=jnp.int32)
            l_chunks = []
            for k in range(MAXLEN // 16):
                l_chunks.append(k * 16)
            l_chunks.append(MAXLEN - 16)

            def dcol(d, carry2):
                cidx_a = jnp.full((16,), 0, jnp.int32) + d
                cidx_b = cidx_a + HIDDEN
                for l0 in l_chunks:
                    if l0 + 16 <= SPLIT_A:
                        ridx = iota16 + l0
                        v = plsc.load_gather(vin[p], [ridx, cidx_a])
                    else:
                        ridx = iota16 + (l0 - SPLIT_A)
                        v = plsc.load_gather(vin[p], [ridx, cidx_b])
                    vout[p][d, pl.ds(l0, 16)] = v
                return carry2

            lax.fori_loop(0, HIDDEN, dcol, 0)
            s4, d4 = out_pair(c, p)
            pltpu.async_copy(s4, d4, osem[p])
        return carry

    lax.fori_loop(0, SEQ_PER_W // CP, outer, 0)
    for k in range(CP):
        c = SEQ_PER_W - CP + k
        s5, d5 = out_pair(c, c % CP)
        pltpu.make_async_copy(s5, d5, osem[c % CP]).wait()


def kernel(item_seq, item_table, pos_table):
    idx_f = item_seq.astype(jnp.float32)
    bridge = _embed_kernel(idx_f, item_table, pos_table)
    return _retile_out(bridge).transpose(0, 2, 1)


# speedup vs baseline: 1.6091x; 1.6091x over previous
import functools

import jax
import jax.numpy as jnp
from jax import lax
from jax.experimental import pallas as pl
from jax.experimental.pallas import tpu as pltpu
from jax.experimental.pallas import tpu_sc as plsc

ITEM_NUM = 1000000
HIDDEN = 64
MAXLEN = 200
BATCH = 4096

SCALE = float(HIDDEN) ** 0.5

NUM_CORES = 2
NUM_SUBCORES = 16
NW = NUM_CORES * NUM_SUBCORES
SEQ_PER_W = BATCH // NW
NB = 4
LA = 2

_mesh = plsc.VectorSubcoreMesh(core_axis_name="c", subcore_axis_name="s")


@functools.partial(
    pl.kernel,
    mesh=_mesh,
    compiler_params=pltpu.CompilerParams(use_tc_tiling_on_sc=False),
    out_type=jax.ShapeDtypeStruct((BATCH, MAXLEN, HIDDEN), jnp.float32),
    scratch_types=(
        [pltpu.VMEM((SEQ_PER_W, MAXLEN), jnp.int32),
         pltpu.VMEM((MAXLEN, HIDDEN), jnp.float32)]
        + [pltpu.VMEM((MAXLEN, HIDDEN), jnp.float32) for _ in range(NB)]
        + [pltpu.SemaphoreType.DMA for _ in range(2 * NB)]
    ),
)
def _embed_kernel(idx_hbm, table_hbm, pos_hbm, out_hbm, idx_v, pos_v,
                  b0, b1, b2, b3, g0, g1, g2, g3, s0, s1, s2, s3):
    bufs = [b0, b1, b2, b3]
    gsem = [g0, g1, g2, g3]
    ssem = [s0, s1, s2, s3]
    wid = lax.axis_index("s") * NUM_CORES + lax.axis_index("c")
    seq_base = wid * SEQ_PER_W
    pltpu.sync_copy(idx_hbm.at[pl.ds(seq_base, SEQ_PER_W)], idx_v)
    pltpu.sync_copy(pos_hbm, pos_v)

    def g_start(c, b):
        pltpu.async_copy(table_hbm.at[idx_v.at[c]], bufs[b], gsem[b])

    def g_wait(c, b):
        pltpu.make_async_copy(table_hbm.at[idx_v.at[c]], bufs[b],
                              gsem[b]).wait()

    def s_start(c, b):
        pltpu.async_copy(bufs[b], out_hbm.at[seq_base + c], ssem[b])

    def s_wait(c, b):
        pltpu.make_async_copy(bufs[b], out_hbm.at[seq_base + c],
                              ssem[b]).wait()

    for c in range(LA):
        g_start(c, c % NB)

    def outer(o, carry):
        for b in range(NB):
            c = o * NB + b
            g_wait(c, b)

            def row(j, carry2):
                for h in range(HIDDEN // 16):
                    sl = pl.ds(h * 16, 16)
                    bufs[b][j, sl] = bufs[b][j, sl] * SCALE + pos_v[j, sl]
                return carry2

            lax.fori_loop(0, MAXLEN, row, 0, unroll=4)
            s_start(c, b)
            b3 = (b + LA + 1) % NB

            @pl.when(c >= 1)
            def _():
                s_wait(c - 1, b3)
            cg = c + LA
            bg = (b + LA) % NB

            @pl.when(cg < SEQ_PER_W)
            def _():
                g_start(cg, bg)
        return carry

    lax.fori_loop(0, SEQ_PER_W // NB, outer, 0)
    c_last = SEQ_PER_W - 1
    s_wait(c_last, c_last % NB)


def kernel(item_seq, item_table, pos_table):
    return _embed_kernel(item_seq, item_table, pos_table)
